# baseline (speedup 1.0000x reference)
import functools

import jax
import jax.numpy as jnp
from jax.experimental import pallas as pl
from jax.experimental.pallas import tpu as pltpu

Q = 1024
D = 128
THRESHOLD = 0.5


def _knn_body(q_ref, k_ref, w_ref, b_ref, label_ref, val_ref, m_scr, i_scr,
              *, kb, nk, nblocks):
    i = pl.program_id(0)

    @pl.when(i == 0)
    def _init():
        m_scr[...] = jnp.full((Q, 1), -jnp.inf, jnp.float32)
        i_scr[...] = jnp.zeros((Q, 1), jnp.int32)

    w = w_ref[0]
    b = b_ref[0]
    sgn = jnp.sign(w)

    q = q_ref[...]
    qn = q / (jnp.sqrt(jnp.sum(q * q, axis=1, keepdims=True)) + 1e-12)
    qn = qn * sgn

    k = k_ref[...]
    kn = k / (jnp.sqrt(jnp.sum(k * k, axis=1, keepdims=True)) + 1e-12)

    s = jax.lax.dot_general(qn.astype(jnp.bfloat16), kn.astype(jnp.bfloat16),
                            (((1,), (1,)), ((), ())),
                            preferred_element_type=jnp.float32)

    if nblocks * kb != nk:
        @pl.when(i == nblocks - 1)
        def _mask():
            col = jax.lax.broadcasted_iota(jnp.int32, s.shape, 1) + i * kb
            s_m = jnp.where(col < nk, s, -jnp.inf)
            _merge(s_m, i, kb, m_scr, i_scr)

        @pl.when(i != nblocks - 1)
        def _nomask():
            _merge(s, i, kb, m_scr, i_scr)
    else:
        _merge(s, i, kb, m_scr, i_scr)

    @pl.when(i == nblocks - 1)
    def _fin():
        sim_win = m_scr[...] * sgn
        vals = jax.nn.sigmoid(w * sim_win + b)
        val_ref[...] = vals
        label_ref[...] = jnp.where(vals >= THRESHOLD, i_scr[...], -1)


def _merge(s, i, kb, m_scr, i_scr):
    bmax = jnp.max(s, axis=1, keepdims=True)
    bidx = jnp.argmax(s, axis=1).astype(jnp.int32).reshape(Q, 1) + i * kb
    run_m = m_scr[...]
    upd = bmax > run_m
    i_scr[...] = jnp.where(upd, bidx, i_scr[...])
    m_scr[...] = jnp.where(upd, bmax, run_m)


def kernel(queries, keys, w, b):
    kb = 2048
    nk = keys.shape[0]
    nblocks = pl.cdiv(nk, kb)
    kpad = nblocks * kb
    if kpad != nk:
        keys = jnp.pad(keys, ((0, kpad - nk), (0, 0)))

    label2, vals2 = pl.pallas_call(
        functools.partial(_knn_body, kb=kb, nk=nk, nblocks=nblocks),
        grid=(nblocks,),
        in_specs=[
            pl.BlockSpec((Q, D), lambda i: (0, 0)),
            pl.BlockSpec((kb, D), lambda i: (i, 0)),
            pl.BlockSpec(memory_space=pltpu.SMEM),
            pl.BlockSpec(memory_space=pltpu.SMEM),
        ],
        out_specs=[
            pl.BlockSpec((Q, 1), lambda i: (0, 0)),
            pl.BlockSpec((Q, 1), lambda i: (0, 0)),
        ],
        out_shape=[
            jax.ShapeDtypeStruct((Q, 1), jnp.int32),
            jax.ShapeDtypeStruct((Q, 1), jnp.float32),
        ],
        scratch_shapes=[
            pltpu.VMEM((Q, 1), jnp.float32),
            pltpu.VMEM((Q, 1), jnp.int32),
        ],
    )(queries, keys, w, b)
    return label2.reshape(-1), vals2.reshape(-1)


# speedup vs baseline: 1.2450x; 1.2450x over previous
import functools

import jax
import jax.numpy as jnp
from jax.experimental import pallas as pl
from jax.experimental.pallas import tpu as pltpu

Q = 1024
D = 128
LANES = 128
THRESHOLD = 0.5
INT32_MAX = jnp.iinfo(jnp.int32).max


def _knn_body(q_ref, k_ref, w_ref, b_ref, label_ref, val_ref,
              qn_scr, m_scr, i_scr, *, kb, nk, nblocks):
    i = pl.program_id(0)
    nchunks = kb // LANES

    w = w_ref[0]
    b = b_ref[0]
    sgn = jnp.sign(w)

    @pl.when(i == 0)
    def _init():
        q = q_ref[...]
        qn = q / (jnp.sqrt(jnp.sum(q * q, axis=1, keepdims=True)) + 1e-12)
        qn_scr[...] = (qn * sgn).astype(jnp.bfloat16)
        m_scr[...] = jnp.full((Q, LANES), -jnp.inf, jnp.float32)
        i_scr[...] = jnp.zeros((Q, LANES), jnp.int32)

    k = k_ref[...]
    kn = k / (jnp.sqrt(jnp.sum(k * k, axis=1, keepdims=True)) + 1e-12)

    s = jax.lax.dot_general(qn_scr[...], kn.astype(jnp.bfloat16),
                            (((1,), (1,)), ((), ())),
                            preferred_element_type=jnp.float32)

    if nblocks * kb != nk:
        @pl.when(i == nblocks - 1)
        def _mask():
            col = jax.lax.broadcasted_iota(jnp.int32, s.shape, 1) + i * kb
            _scan(jnp.where(col < nk, s, -jnp.inf), i, nchunks, m_scr, i_scr)

        @pl.when(i != nblocks - 1)
        def _nomask():
            _scan(s, i, nchunks, m_scr, i_scr)
    else:
        _scan(s, i, nchunks, m_scr, i_scr)

    @pl.when(i == nblocks - 1)
    def _fin():
        m = m_scr[...]
        lane = jax.lax.broadcasted_iota(jnp.int32, (Q, LANES), 1)
        col = i_scr[...] * LANES + lane
        bmax = jnp.max(m, axis=1, keepdims=True)
        cand = jnp.where(m == bmax, col, INT32_MAX)
        bidx = jnp.min(cand, axis=1, keepdims=True)
        sim_win = bmax * sgn
        vals = jax.nn.sigmoid(w * sim_win + b)
        val_ref[...] = vals
        label_ref[...] = jnp.where(vals >= THRESHOLD, bidx, -1)


def _scan(s, i, nchunks, m_scr, i_scr):
    m = m_scr[...]
    t = i_scr[...]
    base = i * nchunks
    for j in range(nchunks):
        c = s[:, j * LANES:(j + 1) * LANES]
        upd = c > m
        t = jnp.where(upd, base + j, t)
        m = jnp.maximum(m, c)
    m_scr[...] = m
    i_scr[...] = t


def kernel(queries, keys, w, b):
    kb = 2048
    nk = keys.shape[0]
    nblocks = pl.cdiv(nk, kb)
    kpad = nblocks * kb
    if kpad != nk:
        keys = jnp.pad(keys, ((0, kpad - nk), (0, 0)))

    label2, vals2 = pl.pallas_call(
        functools.partial(_knn_body, kb=kb, nk=nk, nblocks=nblocks),
        grid=(nblocks,),
        in_specs=[
            pl.BlockSpec((Q, D), lambda i: (0, 0)),
            pl.BlockSpec((kb, D), lambda i: (i, 0)),
            pl.BlockSpec(memory_space=pltpu.SMEM),
            pl.BlockSpec(memory_space=pltpu.SMEM),
        ],
        out_specs=[
            pl.BlockSpec((Q, 1), lambda i: (0, 0)),
            pl.BlockSpec((Q, 1), lambda i: (0, 0)),
        ],
        out_shape=[
            jax.ShapeDtypeStruct((Q, 1), jnp.int32),
            jax.ShapeDtypeStruct((Q, 1), jnp.float32),
        ],
        scratch_shapes=[
            pltpu.VMEM((Q, D), jnp.bfloat16),
            pltpu.VMEM((Q, LANES), jnp.float32),
            pltpu.VMEM((Q, LANES), jnp.int32),
        ],
    )(queries, keys, w, b)
    return label2.reshape(-1), vals2.reshape(-1)
